# baseline (speedup 1.0000x reference)
import jax
import jax.numpy as jnp
from jax.experimental import pallas as pl

VOCAB = 10
DIM = 10
TOK = 200
ROWS = 16384
BR = 64


def _body(x_ref, w1_ref, w2_ref, out_ref):
    n = BR * TOK
    x3 = x_ref[...].reshape(BR, TOK, 1)
    oh = (x3 == jax.lax.broadcasted_iota(jnp.int32, (1, 1, 16), 2))
    oh = oh.astype(jnp.bfloat16).reshape(n, 16)
    wsum = (w1_ref[...] + w2_ref[...]).astype(jnp.bfloat16)
    wp = jnp.concatenate(
        [wsum, jnp.zeros((VOCAB, 128 - DIM), jnp.bfloat16)], axis=1)
    wp = jnp.concatenate([wp, jnp.zeros((6, 128), jnp.bfloat16)], axis=0)
    out = jnp.dot(oh, wp, preferred_element_type=jnp.float32)
    out_ref[...] = out[:, :DIM].reshape(BR, TOK, DIM)


@jax.jit
def kernel(x, W1, W2):
    return pl.pallas_call(
        _body,
        grid=(ROWS // BR,),
        in_specs=[
            pl.BlockSpec((BR, TOK), lambda i: (i, 0)),
            pl.BlockSpec((VOCAB, DIM), lambda i: (0, 0)),
            pl.BlockSpec((VOCAB, DIM), lambda i: (0, 0)),
        ],
        out_specs=pl.BlockSpec((BR, TOK, DIM), lambda i: (i, 0, 0)),
        out_shape=jax.ShapeDtypeStruct((ROWS, TOK, DIM), jnp.float32),
    )(x, W1, W2)


# speedup vs baseline: 16.6827x; 16.6827x over previous
import jax
import jax.numpy as jnp
from jax.experimental import pallas as pl
from jax.experimental.pallas import tpu as pltpu

VOCAB = 10
DIM = 10
TOK = 200
ROWS = 16384
BT = 8


def _body(x_ref, w1_ref, w2_ref, out_ref):
    xb = x_ref[...]
    masks = [xb == v for v in range(VOCAB)]
    for d in range(DIM):
        od = jnp.zeros(xb.shape, jnp.float32)
        for v in range(VOCAB):
            w = w1_ref[v, d] + w2_ref[v, d]
            od = jnp.where(masks[v], w, od)
        out_ref[d] = od


@jax.jit
def kernel(x, W1, W2):
    xt = x.T
    outt = pl.pallas_call(
        _body,
        grid=(TOK // BT,),
        in_specs=[
            pl.BlockSpec((BT, ROWS), lambda i: (i, 0)),
            pl.BlockSpec(memory_space=pltpu.SMEM),
            pl.BlockSpec(memory_space=pltpu.SMEM),
        ],
        out_specs=pl.BlockSpec((DIM, BT, ROWS), lambda i: (0, i, 0)),
        out_shape=jax.ShapeDtypeStruct((DIM, TOK, ROWS), jnp.float32),
    )(xt, W1, W2)
    return outt.transpose(2, 1, 0)


# speedup vs baseline: 18.5752x; 1.1134x over previous
import jax
import jax.numpy as jnp
from jax.experimental import pallas as pl
from jax.experimental.pallas import tpu as pltpu

VOCAB = 10
DIM = 10
TOK = 200
ROWS = 16384
BT = 8


CH = 512


def _body(x_ref, w1_ref, w2_ref, out_ref):
    ws = [[w1_ref[v, d] + w2_ref[v, d] for d in range(DIM)]
          for v in range(VOCAB)]
    for c in range(ROWS // CH):
        sl = slice(c * CH, (c + 1) * CH)
        xc = x_ref[:, sl]
        accs = [jnp.zeros((BT, CH), jnp.float32) for _ in range(DIM)]
        for v in range(VOCAB):
            m = xc == v
            for d in range(DIM):
                accs[d] = jnp.where(m, ws[v][d], accs[d])
        for d in range(DIM):
            out_ref[d, :, sl] = accs[d]


@jax.jit
def kernel(x, W1, W2):
    xt = x.T
    outt = pl.pallas_call(
        _body,
        grid=(TOK // BT,),
        in_specs=[
            pl.BlockSpec((BT, ROWS), lambda i: (i, 0)),
            pl.BlockSpec(memory_space=pltpu.SMEM),
            pl.BlockSpec(memory_space=pltpu.SMEM),
        ],
        out_specs=pl.BlockSpec((DIM, BT, ROWS), lambda i: (0, i, 0)),
        out_shape=jax.ShapeDtypeStruct((DIM, TOK, ROWS), jnp.float32),
    )(xt, W1, W2)
    return outt.transpose(2, 1, 0)


# speedup vs baseline: 26.1480x; 1.4077x over previous
import jax
import jax.numpy as jnp
from jax.experimental import pallas as pl
from jax.experimental.pallas import tpu as pltpu

VOCAB = 10
DIM = 10
TOK = 200
ROWS = 16384
BT = 8


CH = 512


def _body(x_ref, w1_ref, w2_ref, out_ref):
    ws = [[w1_ref[v, d] + w2_ref[v, d] for d in range(DIM)]
          for v in range(VOCAB)]
    for c in range(ROWS // CH):
        sl = slice(c * CH, (c + 1) * CH)
        xc = x_ref[:, sl]
        b0 = (xc & 1) != 0
        b1 = (xc & 2) != 0
        b2 = (xc & 4) != 0
        b3 = xc >= 8
        for d in range(DIM):
            s01 = jnp.where(b0, ws[1][d], ws[0][d])
            s23 = jnp.where(b0, ws[3][d], ws[2][d])
            s45 = jnp.where(b0, ws[5][d], ws[4][d])
            s67 = jnp.where(b0, ws[7][d], ws[6][d])
            s89 = jnp.where(b0, ws[9][d], ws[8][d])
            t03 = jnp.where(b1, s23, s01)
            t47 = jnp.where(b1, s67, s45)
            u07 = jnp.where(b2, t47, t03)
            out_ref[d, :, sl] = jnp.where(b3, s89, u07)


@jax.jit
def kernel(x, W1, W2):
    xt = x.T
    outt = pl.pallas_call(
        _body,
        grid=(TOK // BT,),
        in_specs=[
            pl.BlockSpec((BT, ROWS), lambda i: (i, 0)),
            pl.BlockSpec(memory_space=pltpu.SMEM),
            pl.BlockSpec(memory_space=pltpu.SMEM),
        ],
        out_specs=pl.BlockSpec((DIM, BT, ROWS), lambda i: (0, i, 0)),
        out_shape=jax.ShapeDtypeStruct((DIM, TOK, ROWS), jnp.float32),
    )(xt, W1, W2)
    return outt.transpose(2, 1, 0)
